# baseline (speedup 1.0000x reference)
import functools

import jax
import jax.numpy as jnp
from jax import lax
from jax.experimental import pallas as pl
from jax.experimental.pallas import tpu as pltpu
from jax.experimental.pallas import tpu_sc as plsc

_NC = 2
_NS = 16
_NW = _NC * _NS


@functools.lru_cache(maxsize=None)
def _make_kernel(B: int, D: int, LAB: int, ALI: int, MSK: int):
    assert B % (8 * _NW) == 0
    bpw = B // _NW
    lab_pw = LAB // _NW
    ali_pw = ALI // _NW
    msk_pw = MSK // _NW
    assert LAB % (8 * _NW) == 0 and ALI % (8 * _NW) == 0 and MSK % (8 * _NW) == 0

    mesh = plsc.VectorSubcoreMesh(core_axis_name="c", subcore_axis_name="s")

    @functools.partial(
        pl.kernel,
        out_type=(
            jax.ShapeDtypeStruct((B, D), jnp.float32),
            jax.ShapeDtypeStruct((LAB,), jnp.int32),
            jax.ShapeDtypeStruct((ALI,), jnp.float32),
            jax.ShapeDtypeStruct((MSK,), jnp.int32),
        ),
        mesh=mesh,
        scratch_types=[
            pltpu.VMEM((bpw,), jnp.int32),
            pltpu.VMEM((bpw, D), jnp.float32),
            pltpu.SemaphoreType.DMA,
        ],
        compiler_params=pltpu.CompilerParams(use_tc_tiling_on_sc=False),
    )
    def body(table_hbm, idx_hbm, lab_hbm, ali_hbm, msk_hbm,
             out_hbm, lab_out, ali_out, msk_out,
             idx_v, rows_v, sem):
        wid = lax.axis_index("s") * _NC + lax.axis_index("c")
        base = wid * bpw
        pltpu.sync_copy(idx_hbm.at[pl.ds(base, bpw)], idx_v)
        gather = pltpu.async_copy(table_hbm.at[idx_v], rows_v, sem)
        lb = wid * lab_pw
        ab = wid * ali_pw
        mb = wid * msk_pw
        pltpu.sync_copy(lab_hbm.at[pl.ds(lb, lab_pw)], lab_out.at[pl.ds(lb, lab_pw)])
        pltpu.sync_copy(ali_hbm.at[pl.ds(ab, ali_pw)], ali_out.at[pl.ds(ab, ali_pw)])
        pltpu.sync_copy(msk_hbm.at[pl.ds(mb, msk_pw)], msk_out.at[pl.ds(mb, msk_pw)])
        gather.wait()
        pltpu.sync_copy(rows_v, out_hbm.at[pl.ds(base, bpw)])

    return body


def kernel(input_ids, labels, alibi, attention_mask, embed_table):
    ids = input_ids.reshape(-1).astype(jnp.int32)
    B = ids.shape[0]
    D = embed_table.shape[1]
    lab_flat = labels.reshape(-1).astype(jnp.int32)
    ali_flat = alibi.reshape(-1)
    msk_i32 = attention_mask.reshape(-1).astype(jnp.int32)

    fn = _make_kernel(B, D, lab_flat.shape[0], ali_flat.shape[0],
                      msk_i32.shape[0])
    hidden, lab_o, ali_o, msk_o = fn(embed_table, ids, lab_flat, ali_flat,
                                     msk_i32)

    hidden = hidden.reshape(input_ids.shape + (D,))
    lab_o = lab_o.reshape(labels.shape).astype(labels.dtype)
    ali_o = ali_o.reshape(alibi.shape)
    msk_o = (msk_o != 0).reshape(attention_mask.shape)
    return (hidden, lab_o, ali_o, msk_o)
